# Initial kernel scaffold; baseline (speedup 1.0000x reference)
import jax
import jax.numpy as jnp
from jax.experimental import pallas as pl


def kernel(users, pos_items, adj_rows, adj_cols, adj_vals, user_emb, item_emb, W_gc_0, b_gc_0, W_bi_0, b_bi_0, W_gc_1, b_gc_1, W_bi_1, b_bi_1, W_gc_2, b_gc_2, W_bi_2, b_bi_2):
    raise NotImplementedError("write your pallas kernel here")


# speedup vs baseline: 3.5541x; 3.5541x over previous
import functools

import jax
import jax.numpy as jnp
from jax import lax
from jax.experimental import pallas as pl
from jax.experimental.pallas import tpu as pltpu
from jax.experimental.pallas import tpu_sc as plsc

F32 = jnp.float32
HALF_W = 32
DOUT = 64
CH = 128
ZR = 625


def _make_spmm(n, stride, count):
  per_tile = count // 16
  nfull = per_tile // CH
  rem = per_tile - nfull * CH
  rows_pt = -(-n // (16 * 8)) * 8
  n_acc = rows_pt * 16
  rows_last = n - 15 * rows_pt
  mesh = plsc.VectorSubcoreMesh(core_axis_name="c", subcore_axis_name="s")

  rem16 = -(-rem // 16) * 16 if rem else 0
  scratch = [
      pltpu.VMEM((CH,), jnp.int32),
      pltpu.VMEM((CH,), jnp.int32),
      pltpu.VMEM((CH,), F32),
      pltpu.VMEM((CH, HALF_W), F32),
  ]
  if rem:
    scratch += [
        pltpu.VMEM((rem,), jnp.int32),
        pltpu.VMEM((rem,), jnp.int32),
        pltpu.VMEM((rem16,), F32),
        pltpu.VMEM((rem, HALF_W), F32),
    ]
  zr = max(d for d in range(8, 257, 8) if rows_pt % d == 0)
  scratch += [
      pltpu.VMEM((zr, HALF_W), F32),
      pltpu.VMEM_SHARED((n_acc, HALF_W), F32),
      pltpu.SemaphoreType.DMA,
  ]

  @functools.partial(
      pl.kernel,
      out_type=(jax.ShapeDtypeStruct((n, HALF_W), F32),
                jax.ShapeDtypeStruct((n, HALF_W), F32)),
      mesh=mesh,
      scratch_types=scratch,
      compiler_params=pltpu.CompilerParams(use_tc_tiling_on_sc=False),
  )
  def spmm(rows_h, cols_h, vals_h, t0_h, t1_h, o0_h, o1_h, *sc):
    if rem:
      (rows_v, cols_v, vals_v, g_v,
       rows_r, cols_r, vals_r, g_r, z_v, acc, sem) = sc
    else:
      rows_v, cols_v, vals_v, g_v, z_v, acc, sem = sc
    cid = lax.axis_index("c")
    sid = lax.axis_index("s")

    @pl.loop(0, zr)
    def _(i):
      for cc in range(HALF_W // 16):
        z_v[i, pl.ds(cc * 16, 16)] = jnp.zeros((16,), F32)

    @pl.loop(0, rows_pt // zr)
    def _(j):
      pltpu.sync_copy(z_v, acc.at[pl.ds(sid * rows_pt + j * zr, zr)])
    plsc.subcore_barrier()

    def scale_block(g_b, vals_b, i0, cnt):
      vv = vals_b[pl.ds(i0, 16)]
      for u in range(cnt):
        v = vv[u]
        for cc in range(HALF_W // 16):
          sl = (i0 + u, pl.ds(cc * 16, 16))
          g_b[sl] = g_b[sl] * v

    def do_chunk(tbl_h, base, rows_b, cols_b, vals_b, g_b, m):
      pltpu.sync_copy(rows_h.at[pl.ds(base, m)], rows_b)
      pltpu.sync_copy(cols_h.at[pl.ds(base, m)], cols_b)
      pltpu.sync_copy(vals_h.at[pl.ds(base, m)],
                      vals_b if vals_b.shape[0] == m else vals_b.at[pl.ds(0, m)])
      pltpu.async_copy(tbl_h.at[cols_b], g_b, sem).wait()

      @pl.loop(0, m // 16)
      def _(blk):
        scale_block(g_b, vals_b, blk * 16, 16)
      if m % 16:
        scale_block(g_b, vals_b, (m // 16) * 16, m % 16)

      pltpu.sync_copy(g_b, acc.at[rows_b], add=True)

    for c in range(2):
      @pl.when(cid == c)
      def _(c=c):
        tbl_h = (t0_h, t1_h)[c]
        tb = c * stride + sid * per_tile

        @pl.loop(0, nfull)
        def _(k):
          do_chunk(tbl_h, tb + k * CH, rows_v, cols_v, vals_v, g_v, CH)
        if rem:
          do_chunk(tbl_h, tb + nfull * CH, rows_r, cols_r, vals_r, g_r, rem)

    plsc.subcore_barrier()
    for c in range(2):
      @pl.when(cid == c)
      def _(c=c):
        o_h = (o0_h, o1_h)[c]

        @pl.when(sid < 15)
        def _():
          pltpu.sync_copy(acc.at[pl.ds(sid * rows_pt, rows_pt)],
                          o_h.at[pl.ds(sid * rows_pt, rows_pt)])

        @pl.when(sid == 15)
        def _():
          pltpu.sync_copy(acc.at[pl.ds(15 * rows_pt, rows_last)],
                          o_h.at[pl.ds(15 * rows_pt, rows_last)])

  return spmm


def _make_dense(n, layer0, din, R=1000):
  def body(sa_r, sb_r, *rest):
    if layer0:
      e_r, wg_r, bg_r, wb_r, bb_r, lo_r, hi_r, nm_r = rest
      side = (sa_r[...] + sb_r[...])[:, :din]
      ego = e_r[...]
    else:
      ea_r, eb_r, wg_r, bg_r, wb_r, bb_r, lo_r, hi_r, nm_r = rest
      side = jnp.concatenate([sa_r[...], sb_r[...]], axis=1)
      ego = jnp.concatenate([ea_r[...], eb_r[...]], axis=1)
    sum_e = jnp.dot(side, wg_r[...], preferred_element_type=F32) + bg_r[...]
    bi = jnp.dot(ego * side, wb_r[...], preferred_element_type=F32) + bb_r[...]
    x = sum_e + bi
    x = jnp.where(x >= 0, x, x * 0.2)
    lo_r[...] = x[:, :HALF_W]
    hi_r[...] = x[:, HALF_W:]
    nrm = jnp.sqrt(jnp.sum(x * x, axis=1, keepdims=True))
    nm_r[...] = x / jnp.maximum(nrm, 1e-12)

  row_spec = lambda w: pl.BlockSpec((R, w), lambda i: (i, 0))
  full_spec = lambda a, b: pl.BlockSpec((a, b), lambda i: (0, 0))
  in_specs = [row_spec(HALF_W), row_spec(HALF_W)]
  if layer0:
    in_specs += [row_spec(din)]
  else:
    in_specs += [row_spec(HALF_W), row_spec(HALF_W)]
  in_specs += [full_spec(din, DOUT), full_spec(1, DOUT),
               full_spec(din, DOUT), full_spec(1, DOUT)]
  out_specs = [row_spec(HALF_W), row_spec(HALF_W), row_spec(DOUT)]
  return pl.pallas_call(
      body,
      grid=(n // R,),
      in_specs=in_specs,
      out_specs=out_specs,
      out_shape=[jax.ShapeDtypeStruct((n, HALF_W), F32),
                 jax.ShapeDtypeStruct((n, HALF_W), F32),
                 jax.ShapeDtypeStruct((n, DOUT), F32)],
  )


def _make_batch_gather(n, nb):
  per = nb * 4 // 32
  mesh = plsc.VectorSubcoreMesh(core_axis_name="c", subcore_axis_name="s")

  @functools.partial(
      pl.kernel,
      out_type=(jax.ShapeDtypeStruct((nb, HALF_W), F32),
                jax.ShapeDtypeStruct((nb, DOUT), F32),
                jax.ShapeDtypeStruct((nb, DOUT), F32),
                jax.ShapeDtypeStruct((nb, DOUT), F32)),
      mesh=mesh,
      scratch_types=[
          pltpu.VMEM((per,), jnp.int32),
          pltpu.VMEM((per, HALF_W), F32),
          pltpu.VMEM((per, DOUT), F32),
          pltpu.SemaphoreType.DMA,
      ],
      compiler_params=pltpu.CompilerParams(use_tc_tiling_on_sc=False),
  )
  def gk(idx_h, t0, t1, t2, t3, o0, o1, o2, o3, idx_v, b32, b64, sem):
    cid = lax.axis_index("c")
    sid = lax.axis_index("s")
    wid = sid * 2 + cid
    tsel = wid // 8
    base = (wid % 8) * per
    pltpu.sync_copy(idx_h.at[pl.ds(base, per)], idx_v)
    tables = (t0, t1, t2, t3)
    outs = (o0, o1, o2, o3)
    for tt in range(4):
      @pl.when(tsel == tt)
      def _(tt=tt):
        buf = b32 if tt == 0 else b64
        pltpu.async_copy(tables[tt].at[idx_v], buf, sem).wait()
        pltpu.sync_copy(buf, outs[tt].at[pl.ds(base, per)])

  return gk


def _score(u, i):
  def body(u_r, i_r, o_r):
    o_r[...] = lax.dot_general(u_r[...], i_r[...],
                               (((1,), (1,)), ((), ())),
                               preferred_element_type=F32)

  return pl.pallas_call(
      body,
      out_shape=jax.ShapeDtypeStruct((u.shape[0], i.shape[0]), F32),
  )(u, i)


def kernel(users, pos_items, adj_rows, adj_cols, adj_vals, user_emb, item_emb,
           W_gc_0, b_gc_0, W_bi_0, b_bi_0,
           W_gc_1, b_gc_1, W_bi_1, b_bi_1,
           W_gc_2, b_gc_2, W_bi_2, b_bi_2):
  n_user = user_emb.shape[0]
  n = n_user + item_emb.shape[0]
  nnz = adj_rows.shape[0]
  emb = user_emb.shape[1]

  ego0 = jnp.concatenate([user_emb, item_emb], axis=0)
  ego0p = jnp.pad(ego0, ((0, 0), (0, HALF_W - emb)))

  spmm0 = _make_spmm(n, nnz // 2, nnz // 2)
  spmm1 = _make_spmm(n, 0, nnz)

  p0, p1 = spmm0(adj_rows, adj_cols, adj_vals, ego0p, ego0p)
  lo1, hi1, n1 = _make_dense(n, True, emb)(
      p0, p1, ego0, W_gc_0, b_gc_0, W_bi_0, b_bi_0)

  s0, s1 = spmm1(adj_rows, adj_cols, adj_vals, lo1, hi1)
  dense = _make_dense(n, False, DOUT)
  lo2, hi2, n2 = dense(s0, s1, lo1, hi1, W_gc_1, b_gc_1, W_bi_1, b_bi_1)

  s0, s1 = spmm1(adj_rows, adj_cols, adj_vals, lo2, hi2)
  _, _, n3 = dense(s0, s1, lo2, hi2, W_gc_2, b_gc_2, W_bi_2, b_bi_2)

  b = users.shape[0]
  idx = jnp.concatenate([users, pos_items + n_user]).astype(jnp.int32)
  g0, g1, g2, g3 = _make_batch_gather(n, 2 * b)(idx, ego0p, n1, n2, n3)
  u_g = jnp.concatenate([g0[:b, :emb], g1[:b], g2[:b], g3[:b]], axis=1)
  i_g = jnp.concatenate([g0[b:, :emb], g1[b:], g2[b:], g3[b:]], axis=1)
  return _score(u_g, i_g)


# speedup vs baseline: 10.1032x; 2.8427x over previous
import functools

import jax
import jax.numpy as jnp
from jax import lax
from jax.experimental import pallas as pl
from jax.experimental.pallas import tpu as pltpu
from jax.experimental.pallas import tpu_sc as plsc

F32 = jnp.float32
HALF_W = 32
DOUT = 64
CH = 128
ZR = 625


G = 4


def _make_spmm(n, stride_ch, count_ch):
  rows_pt = -(-n // (16 * 8)) * 8
  n_acc = rows_pt * 16
  rows_last = n - 15 * rows_pt
  tg = -(-count_ch // G)
  zfull = rows_pt // CH
  ztail = rows_pt - zfull * CH
  mesh = plsc.VectorSubcoreMesh(core_axis_name="c", subcore_axis_name="s")

  edge_set = lambda: [pltpu.VMEM((G, CH), jnp.int32),
                      pltpu.VMEM((G, CH), jnp.int32),
                      pltpu.VMEM((G, CH), F32)]
  scratch = (edge_set() + edge_set()
             + [pltpu.VMEM((CH, HALF_W), F32) for _ in range(G)]
             + [pltpu.VMEM_SHARED((n_acc, HALF_W), F32),
                pltpu.SemaphoreType.DMA,
                pltpu.SemaphoreType.DMA,
                pltpu.SemaphoreType.DMA,
                pltpu.SemaphoreType.DMA])

  @functools.partial(
      pl.kernel,
      out_type=(jax.ShapeDtypeStruct((n, HALF_W), F32),
                jax.ShapeDtypeStruct((n, HALF_W), F32)),
      mesh=mesh,
      scratch_types=scratch,
      compiler_params=pltpu.CompilerParams(use_tc_tiling_on_sc=False),
  )
  def spmm(rows_h, cols_h, vals_h, t0_h, t1_h, o0_h, o1_h, *sc):
    ea = sc[0:3]
    eb = sc[3:6]
    g_b = sc[6:6 + G]
    acc, esem_a, esem_b, gsem, ssem = sc[6 + G:]
    cid = lax.axis_index("c")
    sid = lax.axis_index("s")
    c0 = cid * stride_ch

    @pl.loop(0, CH)
    def _(i):
      for cc in range(HALF_W // 16):
        g_b[0][i, pl.ds(cc * 16, 16)] = jnp.zeros((16,), F32)

    @pl.loop(0, zfull)
    def _(j):
      pltpu.sync_copy(g_b[0], acc.at[pl.ds(sid * rows_pt + j * CH, CH)])
    if ztail:
      pltpu.sync_copy(g_b[0].at[pl.ds(0, ztail)],
                      acc.at[pl.ds(sid * rows_pt + zfull * CH, ztail)])
    plsc.subcore_barrier()

    def edge_issue(gidx, bufs, sem):
      bb = jnp.minimum(gidx * G, count_ch - G)
      for src, dst in zip((rows_h, cols_h, vals_h), bufs):
        pltpu.async_copy(src.at[pl.ds(c0 + bb, G)], dst, sem)

    def edge_wait(bufs, sem):
      for src, dst in zip((rows_h, cols_h, vals_h), bufs):
        pltpu.make_async_copy(src.at[pl.ds(0, G)], dst, sem).wait()

    def scale_chunk(g_c, vb, r):
      @pl.loop(0, CH // 16)
      def _(blk):
        i0 = blk * 16
        vv = vb[r, pl.ds(i0, 16)]
        for u in range(16):
          v = vv[u]
          for cc in range(HALF_W // 16):
            sl = (i0 + u, pl.ds(cc * 16, 16))
            g_c[sl] = g_c[sl] * v

    def process_group(tbl_h, gidx, bufs):
      rb, cb, vb = bufs
      bb = jnp.minimum(gidx * G, count_ch - G)
      koff = gidx * G - bb
      for k in range(G):
        @pl.when(gidx * G + k < count_ch)
        def _(k=k):
          pltpu.async_copy(tbl_h.at[cb.at[koff + k]], g_b[k], gsem)
      for k in range(G):
        @pl.when(gidx * G + k < count_ch)
        def _(k=k):
          pltpu.make_async_copy(tbl_h.at[cb.at[koff + k]], g_b[k],
                                gsem).wait()
          scale_chunk(g_b[k], vb, koff + k)
          pltpu.async_copy(g_b[k], acc.at[rb.at[koff + k]], ssem, add=True)
      for k in range(G):
        @pl.when(gidx * G + k < count_ch)
        def _(k=k):
          pltpu.make_async_copy(g_b[k], acc.at[rb.at[koff + k]],
                                ssem).wait()

    for c in range(2):
      @pl.when(cid == c)
      def _(c=c):
        tbl_h = (t0_h, t1_h)[c]
        ngt = (tg - sid + 15) // 16
        npairs = (ngt + 1) // 2
        edge_issue(sid, ea, esem_a)

        @pl.loop(0, npairs)
        def _(i):
          g0 = sid + (2 * i) * 16
          g1 = g0 + 16
          edge_wait(ea, esem_a)

          @pl.when(g1 < tg)
          def _():
            edge_issue(g1, eb, esem_b)
          process_group(tbl_h, g0, ea)

          @pl.when(g1 < tg)
          def _():
            edge_wait(eb, esem_b)

            @pl.when(g1 + 16 < tg)
            def _():
              edge_issue(g1 + 16, ea, esem_a)
            process_group(tbl_h, g1, eb)

    plsc.subcore_barrier()
    for c in range(2):
      @pl.when(cid == c)
      def _(c=c):
        o_h = (o0_h, o1_h)[c]

        @pl.when(sid < 15)
        def _():
          pltpu.sync_copy(acc.at[pl.ds(sid * rows_pt, rows_pt)],
                          o_h.at[pl.ds(sid * rows_pt, rows_pt)])

        @pl.when(sid == 15)
        def _():
          pltpu.sync_copy(acc.at[pl.ds(15 * rows_pt, rows_last)],
                          o_h.at[pl.ds(15 * rows_pt, rows_last)])

  return spmm


def _make_dense(n, layer0, din, R=1000):
  def body(sa_r, sb_r, *rest):
    if layer0:
      e_r, wg_r, bg_r, wb_r, bb_r, lo_r, hi_r, nm_r = rest
      side = (sa_r[...] + sb_r[...])[:, :din]
      ego = e_r[...]
    else:
      ea_r, eb_r, wg_r, bg_r, wb_r, bb_r, lo_r, hi_r, nm_r = rest
      side = jnp.concatenate([sa_r[...], sb_r[...]], axis=1)
      ego = jnp.concatenate([ea_r[...], eb_r[...]], axis=1)
    sum_e = jnp.dot(side, wg_r[...], preferred_element_type=F32) + bg_r[...]
    bi = jnp.dot(ego * side, wb_r[...], preferred_element_type=F32) + bb_r[...]
    x = sum_e + bi
    x = jnp.where(x >= 0, x, x * 0.2)
    lo_r[...] = x[:, :HALF_W]
    hi_r[...] = x[:, HALF_W:]
    nrm = jnp.sqrt(jnp.sum(x * x, axis=1, keepdims=True))
    nm_r[...] = x / jnp.maximum(nrm, 1e-12)

  row_spec = lambda w: pl.BlockSpec((R, w), lambda i: (i, 0))
  full_spec = lambda a, b: pl.BlockSpec((a, b), lambda i: (0, 0))
  in_specs = [row_spec(HALF_W), row_spec(HALF_W)]
  if layer0:
    in_specs += [row_spec(din)]
  else:
    in_specs += [row_spec(HALF_W), row_spec(HALF_W)]
  in_specs += [full_spec(din, DOUT), full_spec(1, DOUT),
               full_spec(din, DOUT), full_spec(1, DOUT)]
  out_specs = [row_spec(HALF_W), row_spec(HALF_W), row_spec(DOUT)]
  return pl.pallas_call(
      body,
      grid=(n // R,),
      in_specs=in_specs,
      out_specs=out_specs,
      out_shape=[jax.ShapeDtypeStruct((n, HALF_W), F32),
                 jax.ShapeDtypeStruct((n, HALF_W), F32),
                 jax.ShapeDtypeStruct((n, DOUT), F32)],
  )


def _make_batch_gather(n, nb):
  per = nb * 4 // 32
  mesh = plsc.VectorSubcoreMesh(core_axis_name="c", subcore_axis_name="s")

  @functools.partial(
      pl.kernel,
      out_type=(jax.ShapeDtypeStruct((nb, HALF_W), F32),
                jax.ShapeDtypeStruct((nb, DOUT), F32),
                jax.ShapeDtypeStruct((nb, DOUT), F32),
                jax.ShapeDtypeStruct((nb, DOUT), F32)),
      mesh=mesh,
      scratch_types=[
          pltpu.VMEM((per,), jnp.int32),
          pltpu.VMEM((per, HALF_W), F32),
          pltpu.VMEM((per, DOUT), F32),
          pltpu.SemaphoreType.DMA,
      ],
      compiler_params=pltpu.CompilerParams(use_tc_tiling_on_sc=False),
  )
  def gk(idx_h, t0, t1, t2, t3, o0, o1, o2, o3, idx_v, b32, b64, sem):
    cid = lax.axis_index("c")
    sid = lax.axis_index("s")
    wid = sid * 2 + cid
    tsel = wid // 8
    base = (wid % 8) * per
    pltpu.sync_copy(idx_h.at[pl.ds(base, per)], idx_v)
    tables = (t0, t1, t2, t3)
    outs = (o0, o1, o2, o3)
    for tt in range(4):
      @pl.when(tsel == tt)
      def _(tt=tt):
        buf = b32 if tt == 0 else b64
        pltpu.async_copy(tables[tt].at[idx_v], buf, sem).wait()
        pltpu.sync_copy(buf, outs[tt].at[pl.ds(base, per)])

  return gk


def _score(u, i):
  def body(u_r, i_r, o_r):
    o_r[...] = lax.dot_general(u_r[...], i_r[...],
                               (((1,), (1,)), ((), ())),
                               preferred_element_type=F32)

  return pl.pallas_call(
      body,
      out_shape=jax.ShapeDtypeStruct((u.shape[0], i.shape[0]), F32),
  )(u, i)


def kernel(users, pos_items, adj_rows, adj_cols, adj_vals, user_emb, item_emb,
           W_gc_0, b_gc_0, W_bi_0, b_bi_0,
           W_gc_1, b_gc_1, W_bi_1, b_bi_1,
           W_gc_2, b_gc_2, W_bi_2, b_bi_2):
  n_user = user_emb.shape[0]
  n = n_user + item_emb.shape[0]
  nnz = adj_rows.shape[0]
  emb = user_emb.shape[1]

  ego0 = jnp.concatenate([user_emb, item_emb], axis=0)
  ego0p = jnp.pad(ego0, ((0, 0), (0, HALF_W - emb)))

  nch = nnz // CH
  rows2 = adj_rows.reshape(nch, CH)
  cols2 = adj_cols.reshape(nch, CH)
  vals2 = adj_vals.reshape(nch, CH)
  spmm0 = _make_spmm(n, nch // 2, nch // 2)
  spmm1 = _make_spmm(n, 0, nch)

  p0, p1 = spmm0(rows2, cols2, vals2, ego0p, ego0p)
  lo1, hi1, n1 = _make_dense(n, True, emb)(
      p0, p1, ego0, W_gc_0, b_gc_0, W_bi_0, b_bi_0)

  s0, s1 = spmm1(rows2, cols2, vals2, lo1, hi1)
  dense = _make_dense(n, False, DOUT)
  lo2, hi2, n2 = dense(s0, s1, lo1, hi1, W_gc_1, b_gc_1, W_bi_1, b_bi_1)

  s0, s1 = spmm1(rows2, cols2, vals2, lo2, hi2)
  _, _, n3 = dense(s0, s1, lo2, hi2, W_gc_2, b_gc_2, W_bi_2, b_bi_2)

  b = users.shape[0]
  idx = jnp.concatenate([users, pos_items + n_user]).astype(jnp.int32)
  g0, g1, g2, g3 = _make_batch_gather(n, 2 * b)(idx, ego0p, n1, n2, n3)
  u_g = jnp.concatenate([g0[:b, :emb], g1[:b], g2[:b], g3[:b]], axis=1)
  i_g = jnp.concatenate([g0[b:, :emb], g1[b:], g2[b:], g3[b:]], axis=1)
  return _score(u_g, i_g)


# speedup vs baseline: 15.5989x; 1.5440x over previous
import functools

import jax
import jax.numpy as jnp
from jax import lax
from jax.experimental import pallas as pl
from jax.experimental.pallas import tpu as pltpu
from jax.experimental.pallas import tpu_sc as plsc

F32 = jnp.float32
HALF_W = 32
DOUT = 64
CH = 128
G = 4


def _make_spmm(n_acc, stride_ch, count_ch):
  rows_pt = n_acc // 16
  tg = -(-count_ch // G)
  zfull = rows_pt // CH
  ztail = rows_pt - zfull * CH
  mesh = plsc.VectorSubcoreMesh(core_axis_name="c", subcore_axis_name="s")

  edge_set = lambda: [pltpu.VMEM((G, CH), jnp.int32),
                      pltpu.VMEM((G, CH), jnp.int32),
                      pltpu.VMEM((G, CH), F32)]
  scratch = (edge_set() + edge_set()
             + [pltpu.VMEM((CH, HALF_W), F32) for _ in range(G)]
             + [pltpu.VMEM_SHARED((n_acc, HALF_W), F32),
                pltpu.SemaphoreType.DMA,
                pltpu.SemaphoreType.DMA,
                pltpu.SemaphoreType.DMA,
                pltpu.SemaphoreType.DMA])

  @functools.partial(
      pl.kernel,
      out_type=(jax.ShapeDtypeStruct((n_acc, HALF_W), F32),
                jax.ShapeDtypeStruct((n_acc, HALF_W), F32)),
      mesh=mesh,
      scratch_types=scratch,
      compiler_params=pltpu.CompilerParams(use_tc_tiling_on_sc=False),
  )
  def spmm(rows_h, cols_h, vals_h, t0_h, t1_h, o0_h, o1_h, *sc):
    ea = sc[0:3]
    eb = sc[3:6]
    g_b = sc[6:6 + G]
    acc, esem_a, esem_b, gsem, ssem = sc[6 + G:]
    cid = lax.axis_index("c")
    sid = lax.axis_index("s")
    c0 = cid * stride_ch

    @pl.loop(0, CH)
    def _(i):
      for cc in range(HALF_W // 16):
        g_b[0][i, pl.ds(cc * 16, 16)] = jnp.zeros((16,), F32)

    @pl.loop(0, zfull)
    def _(j):
      pltpu.sync_copy(g_b[0], acc.at[pl.ds(sid * rows_pt + j * CH, CH)])
    if ztail:
      pltpu.sync_copy(g_b[0].at[pl.ds(0, ztail)],
                      acc.at[pl.ds(sid * rows_pt + zfull * CH, ztail)])
    plsc.subcore_barrier()

    def edge_issue(gidx, bufs, sem):
      bb = jnp.minimum(gidx * G, count_ch - G)
      for src, dst in zip((rows_h, cols_h, vals_h), bufs):
        pltpu.async_copy(src.at[pl.ds(c0 + bb, G)], dst, sem)

    def edge_wait(bufs, sem):
      for src, dst in zip((rows_h, cols_h, vals_h), bufs):
        pltpu.make_async_copy(src.at[pl.ds(0, G)], dst, sem).wait()

    def scale_chunk(g_c, vb, r):
      @pl.loop(0, CH // 16)
      def _(blk):
        i0 = blk * 16
        vv = vb[r, pl.ds(i0, 16)]
        for u in range(16):
          v = vv[u]
          for cc in range(HALF_W // 16):
            sl = (i0 + u, pl.ds(cc * 16, 16))
            g_c[sl] = g_c[sl] * v

    def process_group(tbl_h, gidx, bufs):
      rb, cb, vb = bufs
      bb = jnp.minimum(gidx * G, count_ch - G)
      koff = gidx * G - bb
      for k in range(G):
        @pl.when(gidx * G + k < count_ch)
        def _(k=k):
          pltpu.async_copy(tbl_h.at[cb.at[koff + k]], g_b[k], gsem)
      for k in range(G):
        @pl.when(gidx * G + k < count_ch)
        def _(k=k):
          pltpu.make_async_copy(tbl_h.at[cb.at[koff + k]], g_b[k],
                                gsem).wait()
          scale_chunk(g_b[k], vb, koff + k)
          pltpu.async_copy(g_b[k], acc.at[rb.at[koff + k]], ssem, add=True)
      for k in range(G):
        @pl.when(gidx * G + k < count_ch)
        def _(k=k):
          pltpu.make_async_copy(g_b[k], acc.at[rb.at[koff + k]],
                                ssem).wait()

    for c in range(2):
      @pl.when(cid == c)
      def _(c=c):
        tbl_h = (t0_h, t1_h)[c]
        ngt = (tg - sid + 15) // 16
        npairs = (ngt + 1) // 2
        edge_issue(sid, ea, esem_a)

        @pl.loop(0, npairs)
        def _(i):
          g0 = sid + (2 * i) * 16
          g1 = g0 + 16
          edge_wait(ea, esem_a)

          @pl.when(g1 < tg)
          def _():
            edge_issue(g1, eb, esem_b)
          process_group(tbl_h, g0, ea)

          @pl.when(g1 < tg)
          def _():
            edge_wait(eb, esem_b)

            @pl.when(g1 + 16 < tg)
            def _():
              edge_issue(g1 + 16, ea, esem_a)
            process_group(tbl_h, g1, eb)

    plsc.subcore_barrier()
    for c in range(2):
      @pl.when(cid == c)
      def _(c=c):
        o_h = (o0_h, o1_h)[c]
        pltpu.sync_copy(acc.at[pl.ds(sid * rows_pt, rows_pt)],
                        o_h.at[pl.ds(sid * rows_pt, rows_pt)])

  return spmm


def _lane_groups(x, off):
  return jnp.concatenate([x[:, 64 * j + off:64 * j + off + 32]
                          for j in range(4)], axis=1)


def _make_dense(np4, layer0, r4):
  def body(*refs):
    if layer0:
      (sa_r, sb_r, e_r, wg_r, wb_r, bg_r, bb_r, m_r, mt_r,
       lo_r, hi_r, na_r, nb_r) = refs
      side4 = sa_r[...] + sb_r[...]
      sum_e = jnp.dot(side4, wg_r[...], preferred_element_type=F32)
      bi = jnp.dot(e_r[...] * side4, wb_r[...], preferred_element_type=F32)
    else:
      (sa_r, sb_r, ea_r, eb_r, wgl_r, wgh_r, wbl_r, wbh_r,
       bg_r, bb_r, m_r, mt_r, lo_r, hi_r, na_r, nb_r) = refs
      sa = sa_r[...]
      sb = sb_r[...]
      sum_e = (jnp.dot(sa, wgl_r[...], preferred_element_type=F32)
               + jnp.dot(sb, wgh_r[...], preferred_element_type=F32))
      bi = (jnp.dot(ea_r[...] * sa, wbl_r[...], preferred_element_type=F32)
            + jnp.dot(eb_r[...] * sb, wbh_r[...], preferred_element_type=F32))
    x = sum_e + bi + bg_r[...] + bb_r[...]
    x = jnp.where(x >= 0, x, x * 0.2)
    lo_r[...] = _lane_groups(x, 0)
    hi_r[...] = _lane_groups(x, 32)
    nsq = jnp.dot(x * x, m_r[...], preferred_element_type=F32)
    den = jnp.maximum(jnp.sqrt(nsq), 1e-12)
    denb = jnp.dot(den, mt_r[...], preferred_element_type=F32)
    xn = x / denb
    na_r[...] = _lane_groups(xn, 0)
    nb_r[...] = _lane_groups(xn, 32)

  rs = lambda: pl.BlockSpec((r4, 128), lambda i: (i, 0))
  fs = lambda a, b: pl.BlockSpec((a, b), lambda i: (0, 0))
  nin = 3 if layer0 else 4
  nw = 2 if layer0 else 4
  in_specs = ([rs() for _ in range(nin)]
              + [fs(128, 256) for _ in range(nw)]
              + [fs(1, 256), fs(1, 256), fs(256, 8), fs(8, 256)])
  out_specs = [rs() for _ in range(4)]
  return pl.pallas_call(
      body,
      grid=(np4 // r4,),
      in_specs=in_specs,
      out_specs=out_specs,
      out_shape=[jax.ShapeDtypeStruct((np4, 128), F32) for _ in range(4)],
  )


def _make_batch_gather(n_acc, nb):
  per = CH
  nchunk = nb // per
  mesh = plsc.VectorSubcoreMesh(core_axis_name="c", subcore_axis_name="s")
  ntab = 7
  nunits = ntab * nchunk

  @functools.partial(
      pl.kernel,
      out_type=tuple(jax.ShapeDtypeStruct((nb, HALF_W), F32)
                     for _ in range(ntab)),
      mesh=mesh,
      scratch_types=[
          pltpu.VMEM((per,), jnp.int32),
          pltpu.VMEM((per, HALF_W), F32),
          pltpu.SemaphoreType.DMA,
      ],
      compiler_params=pltpu.CompilerParams(use_tc_tiling_on_sc=False),
  )
  def gk(idx_h, *refs):
    tabs = refs[:ntab]
    outs = refs[ntab:2 * ntab]
    idx_v, buf, sem = refs[2 * ntab:]
    cid = lax.axis_index("c")
    sid = lax.axis_index("s")
    wid = sid * 2 + cid
    for u0 in range(0, nunits, 32):
      u = wid + u0

      @pl.when(u < nunits)
      def _():
        base = (u % nchunk) * per
        pltpu.sync_copy(idx_h.at[pl.ds(base, per)], idx_v)
        for tt in range(ntab):
          @pl.when(u // nchunk == tt)
          def _(tt=tt):
            pltpu.async_copy(tabs[tt].at[idx_v], buf, sem).wait()
            pltpu.sync_copy(buf, outs[tt].at[pl.ds(base, per)])

  return gk


def _score(u, i):
  def body(u_r, i_r, o_r):
    o_r[...] = lax.dot_general(u_r[...], i_r[...],
                               (((1,), (1,)), ((), ())),
                               preferred_element_type=F32)

  return pl.pallas_call(
      body,
      out_shape=jax.ShapeDtypeStruct((u.shape[0], i.shape[0]), F32),
  )(u, i)


def _blockdiag(w):
  return jnp.kron(jnp.eye(4, dtype=F32), w)


def kernel(users, pos_items, adj_rows, adj_cols, adj_vals, user_emb, item_emb,
           W_gc_0, b_gc_0, W_bi_0, b_bi_0,
           W_gc_1, b_gc_1, W_bi_1, b_bi_1,
           W_gc_2, b_gc_2, W_bi_2, b_bi_2):
  n_user = user_emb.shape[0]
  n = n_user + item_emb.shape[0]
  nnz = adj_rows.shape[0]
  emb = user_emb.shape[1]
  n_acc = -(-n // 128) * 128
  np4 = n_acc * HALF_W // 128
  r4 = max(d for d in range(8, 1025, 8) if np4 % d == 0)

  flat = lambda x: jnp.reshape(x, (-1,))
  as_sc = lambda x: jnp.reshape(flat(x), (n_acc, HALF_W))
  as_tc = lambda x: jnp.reshape(flat(x), (np4, 128))

  ego0 = jnp.concatenate([user_emb, item_emb], axis=0)
  ego0p = jnp.pad(ego0, ((0, n_acc - n), (0, HALF_W - emb)))

  nch = nnz // CH
  rows2 = adj_rows.reshape(nch, CH)
  cols2 = adj_cols.reshape(nch, CH)
  vals2 = adj_vals.reshape(nch, CH)
  spmm0 = _make_spmm(n_acc, nch // 2, nch // 2)
  spmm1 = _make_spmm(n_acc, 0, nch)

  wg0 = _blockdiag(jnp.pad(W_gc_0, ((0, HALF_W - emb), (0, 0))))
  wb0 = _blockdiag(jnp.pad(W_bi_0, ((0, HALF_W - emb), (0, 0))))
  wk = {}
  for k, (wg, wb) in ((1, (W_gc_1, W_bi_1)), (2, (W_gc_2, W_bi_2))):
    wk[k] = (_blockdiag(wg[:HALF_W]), _blockdiag(wg[HALF_W:]),
             _blockdiag(wb[:HALF_W]), _blockdiag(wb[HALF_W:]))
  bg = {k: jnp.tile(b, (1, 4)) for k, b in
        ((0, b_gc_0), (1, b_gc_1), (2, b_gc_2))}
  bb = {k: jnp.tile(b, (1, 4)) for k, b in
        ((0, b_bi_0), (1, b_bi_1), (2, b_bi_2))}
  m_ind = jnp.kron(jnp.eye(4, dtype=F32), jnp.ones((DOUT, 1), F32))
  m_ind = jnp.pad(m_ind, ((0, 0), (0, 4)))
  mt_ind = jnp.kron(jnp.eye(4, dtype=F32), jnp.ones((1, DOUT), F32))
  mt_ind = jnp.pad(mt_ind, ((0, 4), (0, 0)))

  dense0 = _make_dense(np4, True, r4)
  dense = _make_dense(np4, False, r4)

  p0, p1 = spmm0(rows2, cols2, vals2, ego0p, ego0p)
  lo1, hi1, n1a, n1b = dense0(as_tc(p0), as_tc(p1), as_tc(ego0p),
                              wg0, wb0, bg[0], bb[0], m_ind, mt_ind)

  s0, s1 = spmm1(rows2, cols2, vals2, as_sc(lo1), as_sc(hi1))
  lo2, hi2, n2a, n2b = dense(as_tc(s0), as_tc(s1), lo1, hi1,
                             *wk[1], bg[1], bb[1], m_ind, mt_ind)

  s0, s1 = spmm1(rows2, cols2, vals2, as_sc(lo2), as_sc(hi2))
  _, _, n3a, n3b = dense(as_tc(s0), as_tc(s1), lo2, hi2,
                         *wk[2], bg[2], bb[2], m_ind, mt_ind)

  b = users.shape[0]
  idx = jnp.concatenate([users, pos_items + n_user]).astype(jnp.int32)
  gs = _make_batch_gather(n_acc, 2 * b)(
      idx, as_sc(ego0p), as_sc(n1a), as_sc(n1b), as_sc(n2a), as_sc(n2b),
      as_sc(n3a), as_sc(n3b))
  pieces = [gs[0][:, :emb]] + list(gs[1:])
  u_g = jnp.concatenate([p[:b] for p in pieces], axis=1)
  i_g = jnp.concatenate([p[b:] for p in pieces], axis=1)
  return _score(u_g, i_g)
